# Initial kernel scaffold; baseline (speedup 1.0000x reference)
import jax
import jax.numpy as jnp
from jax.experimental import pallas as pl


def kernel(x, edge_index, edge_attr, alphas_raw):
    raise NotImplementedError("write your pallas kernel here")


# speedup vs baseline: 7.3206x; 7.3206x over previous
import functools

import jax
import jax.numpy as jnp
from jax import lax
from jax.experimental import pallas as pl
from jax.experimental.pallas import tpu as pltpu
from jax.experimental.pallas import tpu_sc as plsc

_NC = 2
_NS = 16
_LANES = 16
_NW = _NC * _NS
_K = 128


def _deg_sc(row3, w3, n_node):
    nb = row3.shape[1]
    mesh = plsc.VectorSubcoreMesh(core_axis_name="c", subcore_axis_name="s")

    @functools.partial(
        pl.kernel,
        out_type=jax.ShapeDtypeStruct((_NW, n_node), jnp.float32),
        mesh=mesh,
        compiler_params=pltpu.CompilerParams(needs_layout_passes=False),
        scratch_types=[
            pltpu.VMEM((n_node,), jnp.float32),
            pltpu.VMEM((nb, _K), jnp.int32),
            pltpu.VMEM((nb, _K), jnp.float32),
        ],
    )
    def k(row_hbm, w_hbm, out_hbm, dacc, ridx, wv):
        c = lax.axis_index("c")
        s = lax.axis_index("s")
        wid = s * _NC + c
        pltpu.sync_copy(row_hbm.at[wid], ridx)
        pltpu.sync_copy(w_hbm.at[wid], wv)
        zeros = jnp.zeros((_LANES,), jnp.float32)

        def zbody(i, carry):
            dacc[pl.ds(i * _LANES, _LANES)] = zeros
            return carry

        lax.fori_loop(0, n_node // _LANES, zbody, 0)
        spb = _K // _LANES

        def body(i, carry):
            b = i // spb
            j = i % spb
            idx = ridx[b, pl.ds(j * _LANES, _LANES)]
            w = wv[b, pl.ds(j * _LANES, _LANES)]
            plsc.addupdate_scatter(dacc, [idx], w)
            return carry

        lax.fori_loop(0, nb * spb, body, 0)
        pltpu.sync_copy(dacc, out_hbm.at[wid])

    return k(row3, w3)


def _spmm_sc(u, col3, row3, w3, n_node, d):
    nb = col3.shape[1]
    nps = n_node // _NS
    mesh = plsc.VectorSubcoreMesh(core_axis_name="c", subcore_axis_name="s")

    @functools.partial(
        pl.kernel,
        out_type=jax.ShapeDtypeStruct((_NC, n_node, d), jnp.float32),
        mesh=mesh,
        compiler_params=pltpu.CompilerParams(needs_layout_passes=False),
        scratch_types=[
            pltpu.VMEM((nb, _K), jnp.int32),
            pltpu.VMEM((nb, _K), jnp.int32),
            pltpu.VMEM((nb, _K), jnp.float32),
            pltpu.VMEM((_K, d), jnp.float32),
            pltpu.VMEM_SHARED((n_node, d), jnp.float32),
            pltpu.SemaphoreType.DMA,
        ],
    )
    def k(u_hbm, col_hbm, row_hbm, w_hbm, out_hbm,
          cidx, ridx, wv, rows, acc, sem):
        c = lax.axis_index("c")
        s = lax.axis_index("s")
        wid = s * _NC + c
        pltpu.sync_copy(col_hbm.at[wid], cidx)
        pltpu.sync_copy(row_hbm.at[wid], ridx)
        pltpu.sync_copy(w_hbm.at[wid], wv)

        zeros = jnp.zeros((_LANES,), jnp.float32)
        vpr = d // _LANES

        def zrow(i, carry):
            rows[i // vpr, pl.ds((i % vpr) * _LANES, _LANES)] = zeros
            return carry

        lax.fori_loop(0, _K * vpr, zrow, 0)
        nb0 = s * nps
        nfull = nps // _K
        for t in range(nfull):
            pltpu.sync_copy(rows, acc.at[pl.ds(nb0 + t * _K, _K)])
        rem = nps - nfull * _K
        if rem:
            pltpu.sync_copy(rows.at[pl.ds(0, rem)],
                            acc.at[pl.ds(nb0 + nfull * _K, rem)])
        plsc.subcore_barrier()

        def blk(b, carry):
            pltpu.async_copy(u_hbm.at[cidx.at[b]], rows, sem).wait()

            def scale(g, c2):
                wvec = wv[b, pl.ds(g * _LANES, _LANES)]
                for t in range(_LANES):
                    e = g * _LANES + t
                    w = wvec[t]
                    for j in range(vpr):
                        rows[e, pl.ds(j * _LANES, _LANES)] = (
                            rows[e, pl.ds(j * _LANES, _LANES)] * w)
                return c2

            lax.fori_loop(0, _K // _LANES, scale, 0)
            pltpu.sync_copy(rows, acc.at[ridx.at[b]], add=True)
            return carry

        lax.fori_loop(0, nb, blk, 0)
        plsc.subcore_barrier()
        pltpu.sync_copy(acc.at[pl.ds(nb0, nps)],
                        out_hbm.at[c, pl.ds(nb0, nps)])

    return k(u, col3, row3, w3)


def _prep_tc(dp_t, x, araw_page):
    n, d = x.shape

    def body(dp_ref, x_ref, a_ref, u_ref, dinv_ref, al_ref):
        deg = jnp.sum(dp_ref[...], axis=1, keepdims=True)
        pos = deg > 0.0
        dinv = jnp.where(pos, lax.rsqrt(jnp.where(pos, deg, 1.0)), 0.0)
        dinv_ref[...] = dinv
        u_ref[...] = dinv * x_ref[...]
        al_ref[...] = jnp.tanh(a_ref[...])

    return pl.pallas_call(
        body,
        out_shape=(
            jax.ShapeDtypeStruct((n, d), jnp.float32),
            jax.ShapeDtypeStruct((n, 1), jnp.float32),
            jax.ShapeDtypeStruct(araw_page.shape, jnp.float32),
        ),
    )(dp_t, x, araw_page)


def _combine_tc(alpha_page, p0, p1, dinv):
    n, d = p0.shape

    def body(a_ref, p0_ref, p1_ref, di_ref, xl_ref, ul_ref):
        xl = a_ref[...] * (di_ref[...] * (p0_ref[...] + p1_ref[...]))
        xl_ref[...] = xl
        ul_ref[...] = di_ref[...] * xl

    return pl.pallas_call(
        body,
        out_shape=(
            jax.ShapeDtypeStruct((n, d), jnp.float32),
            jax.ShapeDtypeStruct((n, d), jnp.float32),
        ),
    )(alpha_page, p0, p1, dinv)


def kernel(x, edge_index, edge_attr, alphas_raw):
    n, d = x.shape
    e = edge_index.shape[1]
    depth = alphas_raw.shape[0] - 1

    n_pad = -(-n // (_NS * _K)) * (_NS * _K)
    xp = jnp.pad(x, ((0, n_pad - n), (0, 0)))

    ept = -(-e // (_NW * _K)) * _K
    e_pad = ept * _NW
    pad = e_pad - e
    row = jnp.concatenate([edge_index[0], jnp.zeros((pad,), jnp.int32)])
    col = jnp.concatenate([edge_index[1], jnp.zeros((pad,), jnp.int32)])
    w = jnp.concatenate([edge_attr.astype(jnp.float32),
                         jnp.zeros((pad,), jnp.float32)])
    nb = ept // _K
    row3 = row.reshape(_NW, nb, _K)
    col3 = col.reshape(_NW, nb, _K)
    w3 = w.reshape(_NW, nb, _K)

    dp = _deg_sc(row3, w3, n_pad)
    araw_page = jnp.zeros((1, d), jnp.float32).at[0, :depth + 1].set(alphas_raw)
    u, dinv, alphas = _prep_tc(dp.T, xp, araw_page)

    xs = [x]
    for layer in range(1, depth + 1):
        p = _spmm_sc(u, col3, row3, w3, n_pad, d)
        alpha_page = jnp.broadcast_to(alphas[0:1, layer:layer + 1], (1, d))
        xl, u = _combine_tc(alpha_page, p[0], p[1], dinv)
        xs.append(xl[:n])
    return jnp.stack(xs, axis=1)
